# baseline (speedup 1.0000x reference)
import jax
import jax.numpy as jnp
from jax.experimental import pallas as pl
from jax.experimental.pallas import tpu as pltpu

MAX_LEN = 8192


def _add_kernel(x_ref, t_ref, o_ref):
    o_ref[...] = x_ref[...] + t_ref[...]


def kernel(x, table):
    bsz, seq_len, d = x.shape
    if seq_len > MAX_LEN:
        x = x[:, -MAX_LEN:, :]
        seq_len = MAX_LEN
    x2 = x.reshape(bsz * seq_len, d)
    bs = 1024
    nj = seq_len // bs
    out = pl.pallas_call(
        _add_kernel,
        grid=(nj, bsz),
        in_specs=[
            pl.BlockSpec((bs, d), lambda j, b: (b * nj + j, 0)),
            pl.BlockSpec((bs, d), lambda j, b: (j, 0)),
        ],
        out_specs=pl.BlockSpec((bs, d), lambda j, b: (b * nj + j, 0)),
        out_shape=jax.ShapeDtypeStruct(x2.shape, x2.dtype),
        compiler_params=pltpu.CompilerParams(vmem_limit_bytes=60 * 1024 * 1024),
    )(x2, table)
    return out.reshape(bsz, seq_len, d)


# speedup vs baseline: 1.0144x; 1.0144x over previous
import jax
import jax.numpy as jnp
from jax.experimental import pallas as pl
from jax.experimental.pallas import tpu as pltpu

MAX_LEN = 8192


def _add_kernel(x_ref, t_ref, o_ref):
    o_ref[...] = x_ref[...] + t_ref[...]


def kernel(x, table):
    bsz, seq_len, d = x.shape
    if seq_len > MAX_LEN:
        x = x[:, -MAX_LEN:, :]
        seq_len = MAX_LEN
    bs = 512
    bb = bsz // 2
    grid = (seq_len // bs, 2)
    return pl.pallas_call(
        _add_kernel,
        grid=grid,
        in_specs=[
            pl.BlockSpec((bb, bs, d), lambda j, i: (i, j, 0)),
            pl.BlockSpec((bs, d), lambda j, i: (j, 0)),
        ],
        out_specs=pl.BlockSpec((bb, bs, d), lambda j, i: (i, j, 0)),
        out_shape=jax.ShapeDtypeStruct(x.shape, x.dtype),
        compiler_params=pltpu.CompilerParams(vmem_limit_bytes=60 * 1024 * 1024),
    )(x, table)


# speedup vs baseline: 1.0313x; 1.0167x over previous
import jax
import jax.numpy as jnp
from jax.experimental import pallas as pl
from jax.experimental.pallas import tpu as pltpu

MAX_LEN = 8192


def _add_kernel(x_ref, t_ref, o_ref):
    o_ref[...] = x_ref[...] + t_ref[...]


def kernel(x, table):
    bsz, seq_len, d = x.shape
    if seq_len > MAX_LEN:
        x = x[:, -MAX_LEN:, :]
        seq_len = MAX_LEN
    bs = 512
    grid = (seq_len // bs,)
    return pl.pallas_call(
        _add_kernel,
        grid=grid,
        in_specs=[
            pl.BlockSpec((bsz, bs, d), lambda j: (0, j, 0)),
            pl.BlockSpec((bs, d), lambda j: (j, 0)),
        ],
        out_specs=pl.BlockSpec((bsz, bs, d), lambda j: (0, j, 0)),
        out_shape=jax.ShapeDtypeStruct(x.shape, x.dtype),
        compiler_params=pltpu.CompilerParams(vmem_limit_bytes=60 * 1024 * 1024),
    )(x, table)
